# baseline (device time: 19787 ns/iter reference)
import jax
import jax.numpy as jnp
from jax import lax
from jax.experimental import pallas as pl
from jax.experimental.pallas import tpu as pltpu

N_DEV = 4
B = 2
SQ_LOC = 128
SKV_LOC = 128
HQ = 4
DH = 64
D_MODEL = 512
WINDOW = 128
SKV_GLOBAL = N_DEV * SKV_LOC
NEG_INF = -1e9


def kernel(x, Wq, K_ext, V_ext, Wo):
    def body(x_ref, wq_ref, k_ref, v_ref, wo_ref, out_ref,
             kbuf, vbuf, send_sems, recv_sems):
        my = lax.axis_index("i")
        left = (my + N_DEV - 1) % N_DEV
        right = (my + 1) % N_DEV

        barrier_sem = pltpu.get_barrier_semaphore()
        for nbr in (left, right):
            pl.semaphore_signal(
                barrier_sem, inc=1,
                device_id=(nbr,), device_id_type=pl.DeviceIdType.MESH,
            )
        pl.semaphore_wait(barrier_sem, 2)

        kbuf[1] = k_ref[...].astype(jnp.bfloat16)
        vbuf[1] = v_ref[...].astype(jnp.bfloat16)

        rdmas = []
        sem_i = 0
        for dst_slot, nbr in ((2, left), (0, right)):
            for buf in (kbuf, vbuf):
                r = pltpu.make_async_remote_copy(
                    src_ref=buf.at[1],
                    dst_ref=buf.at[dst_slot],
                    send_sem=send_sems.at[sem_i],
                    recv_sem=recv_sems.at[sem_i],
                    device_id=(nbr,),
                    device_id_type=pl.DeviceIdType.MESH,
                )
                r.start()
                rdmas.append(r)
                sem_i += 1

        xb = x_ref[...].astype(jnp.bfloat16).reshape(B * SQ_LOC, D_MODEL)
        wq = wq_ref[...].astype(jnp.bfloat16)
        q2d = jnp.dot(xb, wq, preferred_element_type=jnp.float32)
        q2d = q2d.astype(jnp.bfloat16)

        qi = my * SQ_LOC + lax.broadcasted_iota(
            jnp.int32, (SQ_LOC, 3 * SKV_LOC), 0)
        kj = (my - 1) * SKV_LOC + lax.broadcasted_iota(
            jnp.int32, (SQ_LOC, 3 * SKV_LOC), 1)
        valid = (jnp.abs(qi - kj) <= WINDOW) & (kj >= 0) & (kj < SKV_GLOBAL)

        for r in rdmas:
            r.wait()

        wo = wo_ref[...].astype(jnp.bfloat16)
        for b in range(B):
            head_ctx = []
            for h in range(HQ):
                qbh = q2d[b * SQ_LOC:(b + 1) * SQ_LOC,
                          h * DH:(h + 1) * DH]
                s_slots = [
                    lax.dot_general(
                        qbh, kbuf[s, b, :, h, :],
                        (((1,), (1,)), ((), ())),
                        preferred_element_type=jnp.float32,
                    )
                    for s in range(3)
                ]
                s = jnp.concatenate(s_slots, axis=1) * 0.125
                s = jnp.where(valid, s, NEG_INF)
                m = jnp.max(s, axis=1, keepdims=True)
                w = jnp.exp(s - m)
                w = (w / jnp.sum(w, axis=1, keepdims=True)).astype(jnp.bfloat16)
                vcat = jnp.concatenate(
                    [vbuf[s2, b, :, h, :] for s2 in range(3)], axis=0)
                ctx = jnp.dot(w, vcat,
                              preferred_element_type=jnp.float32)
                head_ctx.append(ctx)
            ctx_b = jnp.concatenate(head_ctx, axis=1).astype(jnp.bfloat16)
            out_ref[b] = jnp.dot(ctx_b, wo,
                                 preferred_element_type=jnp.float32)

    return pl.pallas_call(
        body,
        out_shape=jax.ShapeDtypeStruct((B, SQ_LOC, D_MODEL), jnp.float32),
        in_specs=[pl.BlockSpec(memory_space=pltpu.VMEM)] * 5,
        out_specs=pl.BlockSpec(memory_space=pltpu.VMEM),
        scratch_shapes=[
            pltpu.VMEM((3, B, SKV_LOC, HQ, DH), jnp.bfloat16),
            pltpu.VMEM((3, B, SKV_LOC, HQ, DH), jnp.bfloat16),
            pltpu.SemaphoreType.DMA((4,)),
            pltpu.SemaphoreType.DMA((4,)),
        ],
        compiler_params=pltpu.CompilerParams(collective_id=0),
    )(x, Wq, K_ext, V_ext, Wo)


# device time: 14619 ns/iter; 1.3535x vs baseline; 1.3535x over previous
import jax
import jax.numpy as jnp
from jax import lax
from jax.experimental import pallas as pl
from jax.experimental.pallas import tpu as pltpu

N_DEV = 4
B = 2
SQ_LOC = 128
SKV_LOC = 128
HQ = 4
DH = 64
D_MODEL = 512
WINDOW = 128
SKV_GLOBAL = N_DEV * SKV_LOC
NEG_INF = -1e9


def kernel(x, Wq, K_ext, V_ext, Wo):
    def body(x_ref, wq_ref, k_ref, v_ref, wo_ref, out_ref,
             kbuf, vbuf, send_sems, recv_sems):
        my = lax.axis_index("i")
        left = (my + N_DEV - 1) % N_DEV
        right = (my + 1) % N_DEV

        barrier_sem = pltpu.get_barrier_semaphore()
        for nbr in (left, right):
            pl.semaphore_signal(
                barrier_sem, inc=1,
                device_id=(nbr,), device_id_type=pl.DeviceIdType.MESH,
            )

        def halo_rdmas(buf, sem_base):
            rs = []
            for i, (dst_slot, nbr) in enumerate(((2, left), (0, right))):
                r = pltpu.make_async_remote_copy(
                    src_ref=buf.at[1],
                    dst_ref=buf.at[dst_slot],
                    send_sem=send_sems.at[sem_base + i],
                    recv_sem=recv_sems.at[sem_base + i],
                    device_id=(nbr,),
                    device_id_type=pl.DeviceIdType.MESH,
                )
                r.start()
                rs.append(r)
            return rs

        for b in range(B):
            for h in range(HQ):
                kbuf[1, b, h] = k_ref[b, :, h, :].astype(jnp.bfloat16)
        pl.semaphore_wait(barrier_sem, 2)
        k_rdmas = halo_rdmas(kbuf, 0)
        for b in range(B):
            for h in range(HQ):
                vbuf[1, b, h] = v_ref[b, :, h, :].astype(jnp.bfloat16)
        v_rdmas = halo_rdmas(vbuf, 2)

        xb = x_ref[...].astype(jnp.bfloat16).reshape(B * SQ_LOC, D_MODEL)
        wq = wq_ref[...].astype(jnp.bfloat16)
        q2d = jnp.dot(xb, wq, preferred_element_type=jnp.float32)
        q2d = (q2d * 0.125).astype(jnp.bfloat16)

        qi = my * SQ_LOC + lax.broadcasted_iota(
            jnp.int32, (SQ_LOC, 3 * SKV_LOC), 0)
        kj = (my - 1) * SKV_LOC + lax.broadcasted_iota(
            jnp.int32, (SQ_LOC, 3 * SKV_LOC), 1)
        valid = (jnp.abs(qi - kj) <= WINDOW) & (kj >= 0) & (kj < SKV_GLOBAL)

        wo = wo_ref[...].astype(jnp.bfloat16)

        for r in k_rdmas:
            r.wait()

        ws = []
        for b in range(B):
            for h in range(HQ):
                qbh = q2d[b * SQ_LOC:(b + 1) * SQ_LOC,
                          h * DH:(h + 1) * DH]
                kcat = kbuf[:, b, h].reshape(3 * SKV_LOC, DH)
                s = lax.dot_general(
                    qbh, kcat, (((1,), (1,)), ((), ())),
                    preferred_element_type=jnp.float32,
                )
                u = jnp.exp(jnp.where(valid, s, NEG_INF))
                ws.append((u.astype(jnp.bfloat16),
                           jnp.sum(u, axis=1, keepdims=True)))

        for r in v_rdmas:
            r.wait()

        ctx_rows = []
        for b in range(B):
            head_ctx = []
            for h in range(HQ):
                vcat = vbuf[:, b, h].reshape(3 * SKV_LOC, DH)
                u, sigma = ws[b * HQ + h]
                ctx = jnp.dot(u, vcat,
                              preferred_element_type=jnp.float32)
                head_ctx.append(ctx / sigma)
            ctx_rows.append(jnp.concatenate(head_ctx, axis=1))
        ctx_all = jnp.concatenate(ctx_rows, axis=0).astype(jnp.bfloat16)
        out = jnp.dot(ctx_all, wo, preferred_element_type=jnp.float32)
        out_ref[...] = out.reshape(B, SQ_LOC, D_MODEL)

    return pl.pallas_call(
        body,
        out_shape=jax.ShapeDtypeStruct((B, SQ_LOC, D_MODEL), jnp.float32),
        in_specs=[pl.BlockSpec(memory_space=pltpu.VMEM)] * 5,
        out_specs=pl.BlockSpec(memory_space=pltpu.VMEM),
        scratch_shapes=[
            pltpu.VMEM((3, B, HQ, SKV_LOC, DH), jnp.bfloat16),
            pltpu.VMEM((3, B, HQ, SKV_LOC, DH), jnp.bfloat16),
            pltpu.SemaphoreType.DMA((4,)),
            pltpu.SemaphoreType.DMA((4,)),
        ],
        compiler_params=pltpu.CompilerParams(collective_id=0),
    )(x, Wq, K_ext, V_ext, Wo)
